# baseline (speedup 1.0000x reference)
import functools

import jax
import jax.numpy as jnp
from jax import lax
from jax.experimental import pallas as pl
from jax.experimental.pallas import tpu as pltpu
from jax.experimental.pallas import tpu_sc as plsc

E = 8
TOPK = 2
D_MODEL = 768
D_FF = 768
T = 2048
BLK = 128
KB13 = D_MODEL // BLK
KB2 = D_FF // BLK

TILE_M = 256
A = T * TOPK
MAX_TILES = A // TILE_M + E
MAX_ROWS = MAX_TILES * TILE_M

NC = 2
NS = 16
NW = NC * NS
TPW = T // NW
LANES = 16


def _dispatch_body(x_hbm, slot_hbm, xg_hbm, x_v, slot_v, sem):
    wid = lax.axis_index("s") * NC + lax.axis_index("c")
    base = wid * TPW
    pltpu.sync_copy(slot_hbm.at[wid], slot_v)
    pltpu.sync_copy(x_hbm.at[pl.ds(base, TPW)], x_v)
    c0 = pltpu.async_copy(x_v, xg_hbm.at[slot_v.at[0]], sem)
    c0.wait()
    c1 = pltpu.async_copy(x_v, xg_hbm.at[slot_v.at[1]], sem)
    c1.wait()


def _sc_dispatch(x, slot3):
    mesh = plsc.VectorSubcoreMesh(core_axis_name="c", subcore_axis_name="s")
    return pl.kernel(
        _dispatch_body,
        mesh=mesh,
        out_type=jax.ShapeDtypeStruct((MAX_ROWS, D_MODEL), jnp.float32),
        scratch_types=[
            pltpu.VMEM((TPW, D_MODEL), jnp.float32),
            pltpu.VMEM((TOPK, TPW), jnp.int32),
            pltpu.SemaphoreType.DMA,
        ],
    )(x, slot3)


def _combine_body(ys_hbm, slot_hbm, twe_hbm, out_hbm, slot_v, twe_v, a_v, b_v,
                  sem):
    wid = lax.axis_index("s") * NC + lax.axis_index("c")
    base = wid * TPW
    pltpu.sync_copy(slot_hbm.at[wid], slot_v)
    pltpu.sync_copy(twe_hbm.at[pl.ds(base, TPW)], twe_v)
    ca = pltpu.async_copy(ys_hbm.at[slot_v.at[0]], a_v, sem)
    ca.wait()
    cb = pltpu.async_copy(ys_hbm.at[slot_v.at[1]], b_v, sem)
    cb.wait()

    def row(j, carry):
        w0 = twe_v[j, 0, :]
        w1 = twe_v[j, 1, :]
        for c in range(D_MODEL // LANES):
            sl = pl.ds(c * LANES, LANES)
            a_v[j, sl] = a_v[j, sl] * w0 + b_v[j, sl] * w1
        return carry

    lax.fori_loop(0, TPW, row, 0)
    pltpu.sync_copy(a_v, out_hbm.at[pl.ds(base, TPW)])


def _sc_combine(ys, slot3, twe):
    mesh = plsc.VectorSubcoreMesh(core_axis_name="c", subcore_axis_name="s")
    return pl.kernel(
        _combine_body,
        mesh=mesh,
        out_type=jax.ShapeDtypeStruct((T, D_MODEL), jnp.float32),
        scratch_types=[
            pltpu.VMEM((TOPK, TPW), jnp.int32),
            pltpu.VMEM((TPW, TOPK, LANES), jnp.float32),
            pltpu.VMEM((TPW, D_MODEL), jnp.float32),
            pltpu.VMEM((TPW, D_MODEL), jnp.float32),
            pltpu.SemaphoreType.DMA,
        ],
    )(ys, slot3, twe)


def _gmm_body(te_ref, nv_ref, xg_ref, w13_ref, s13_ref, w2_ref, s2_ref,
              ys_ref, w13d_ref, w2d_ref):
    t = pl.program_id(0)

    @pl.when(t < nv_ref[0])
    def _run():
        changed = jnp.logical_or(
            t == 0, te_ref[t] != te_ref[jnp.maximum(t - 1, 0)])

        @pl.when(changed)
        def _dequant():
            for kb in range(KB13):
                sl = pl.ds(kb * BLK, BLK)
                w13d_ref[:, sl] = (w13_ref[0, :, sl]
                                   * s13_ref[0, kb, :][:, None]).astype(jnp.bfloat16)
            for kb in range(KB2):
                sl = pl.ds(kb * BLK, BLK)
                w2d_ref[:, sl] = (w2_ref[0, :, sl]
                                  * s2_ref[0, kb, :][:, None]).astype(jnp.bfloat16)

        xt = xg_ref[...].astype(jnp.bfloat16)
        h = lax.dot_general(xt, w13d_ref[...], (((1,), (1,)), ((), ())),
                            preferred_element_type=jnp.float32)
        gate = h[:, :D_FF]
        up = h[:, D_FF:]
        act = (gate / (1.0 + jnp.exp(-gate)) * up).astype(jnp.bfloat16)
        ys_ref[...] = lax.dot_general(act, w2d_ref[...], (((1,), (1,)), ((), ())),
                                      preferred_element_type=jnp.float32)


def _tc_gmm(te, nv, xg, w13, s13e, w2, s2e):
    return pl.pallas_call(
        _gmm_body,
        grid_spec=pltpu.PrefetchScalarGridSpec(
            num_scalar_prefetch=2,
            grid=(MAX_TILES,),
            in_specs=[
                pl.BlockSpec((TILE_M, D_MODEL), lambda t, te, nv: (t, 0)),
                pl.BlockSpec((1, 2 * D_FF, D_MODEL),
                             lambda t, te, nv: (te[t], 0, 0)),
                pl.BlockSpec((1, KB13, 2 * D_FF),
                             lambda t, te, nv: (te[t], 0, 0)),
                pl.BlockSpec((1, D_MODEL, D_FF),
                             lambda t, te, nv: (te[t], 0, 0)),
                pl.BlockSpec((1, KB2, D_MODEL),
                             lambda t, te, nv: (te[t], 0, 0)),
            ],
            out_specs=pl.BlockSpec((TILE_M, D_MODEL), lambda t, te, nv: (t, 0)),
            scratch_shapes=[
                pltpu.VMEM((2 * D_FF, D_MODEL), jnp.bfloat16),
                pltpu.VMEM((D_MODEL, D_FF), jnp.bfloat16),
            ],
        ),
        out_shape=jax.ShapeDtypeStruct((MAX_ROWS, D_MODEL), jnp.float32),
    )(te, nv, xg, w13, s13e, w2, s2e)


@jax.jit
def _moe_routed(x, topk_ids, topk_weights, w13_fp8, s13e, w2_fp8, s2e):
    e_flat = topk_ids.reshape(-1)
    oh = (e_flat[:, None] == jnp.arange(E, dtype=jnp.int32)).astype(jnp.int32)
    cum = jnp.cumsum(oh, axis=0)
    counts = cum[-1]
    rank = jnp.take_along_axis(cum - oh, e_flat[:, None], axis=1)[:, 0]
    ntiles_e = (counts + TILE_M - 1) // TILE_M
    tile_end = jnp.cumsum(ntiles_e)
    row_off = (tile_end - ntiles_e) * TILE_M
    slot = (row_off[e_flat] + rank).reshape(T, TOPK)
    n_valid = tile_end[-1]
    tidx = jnp.arange(MAX_TILES, dtype=jnp.int32)
    te_raw = jnp.sum((tidx[:, None] >= tile_end[None, :]).astype(jnp.int32),
                     axis=1)
    last_e = te_raw[jnp.maximum(n_valid - 1, 0)]
    te = jnp.where(tidx < n_valid, te_raw, last_e).astype(jnp.int32)
    nv = n_valid.astype(jnp.int32).reshape(1)

    slot3 = slot.reshape(NW, TPW, TOPK).transpose(0, 2, 1)
    twe = jnp.broadcast_to(topk_weights[:, :, None], (T, TOPK, LANES))

    xg = _sc_dispatch(x, slot3)
    ys = _tc_gmm(te, nv, xg, w13_fp8, s13e, w2_fp8, s2e)
    return _sc_combine(ys, slot3, jnp.asarray(twe, jnp.float32))


def kernel(x, topk_ids, topk_weights, moe_n_slice, n_expert_slice, ep_shift,
           w13_fp8, w13_scale_inv, w2_fp8, w2_scale_inv):
    s13e = jnp.repeat(w13_scale_inv.transpose(0, 2, 1), BLK, axis=2)
    s2e = jnp.repeat(w2_scale_inv.transpose(0, 2, 1), BLK, axis=2)
    return _moe_routed(x, topk_ids.astype(jnp.int32),
                       topk_weights.astype(jnp.float32),
                       w13_fp8, s13e, w2_fp8, s2e)


# speedup vs baseline: 1.1047x; 1.1047x over previous
import functools

import jax
import jax.numpy as jnp
from jax import lax
from jax.experimental import pallas as pl
from jax.experimental.pallas import tpu as pltpu
from jax.experimental.pallas import tpu_sc as plsc

E = 8
TOPK = 2
D_MODEL = 768
D_FF = 768
T = 2048
BLK = 128
KB13 = D_MODEL // BLK
KB2 = D_FF // BLK

TILE_M = 256
A = T * TOPK
MAX_TILES = A // TILE_M + E
MAX_ROWS = MAX_TILES * TILE_M

NC = 2
NS = 16
NW = NC * NS
TPW = T // NW
LANES = 16


def _dispatch_body(x_hbm, slot_hbm, xg_hbm, x_v, slot_v, sem):
    wid = lax.axis_index("s") * NC + lax.axis_index("c")
    base = wid * TPW
    pltpu.sync_copy(slot_hbm.at[wid], slot_v)
    pltpu.sync_copy(x_hbm.at[pl.ds(base, TPW)], x_v)
    c0 = pltpu.async_copy(x_v, xg_hbm.at[slot_v.at[0]], sem)
    c0.wait()
    c1 = pltpu.async_copy(x_v, xg_hbm.at[slot_v.at[1]], sem)
    c1.wait()


def _sc_dispatch(x, slot3):
    mesh = plsc.VectorSubcoreMesh(core_axis_name="c", subcore_axis_name="s")
    return pl.kernel(
        _dispatch_body,
        mesh=mesh,
        out_type=jax.ShapeDtypeStruct((MAX_ROWS, D_MODEL), jnp.float32),
        scratch_types=[
            pltpu.VMEM((TPW, D_MODEL), jnp.float32),
            pltpu.VMEM((TOPK, TPW), jnp.int32),
            pltpu.SemaphoreType.DMA,
        ],
    )(x, slot3)


def _combine_body(ys_hbm, slot_hbm, twe_hbm, out_hbm, slot_v, twe_v, a_v, b_v,
                  sem):
    wid = lax.axis_index("s") * NC + lax.axis_index("c")
    base = wid * TPW
    pltpu.sync_copy(slot_hbm.at[wid], slot_v)
    pltpu.sync_copy(twe_hbm.at[pl.ds(base, TPW)], twe_v)
    ca = pltpu.async_copy(ys_hbm.at[slot_v.at[0]], a_v, sem)
    ca.wait()
    cb = pltpu.async_copy(ys_hbm.at[slot_v.at[1]], b_v, sem)
    cb.wait()

    def row(j, carry):
        w0 = twe_v[j, 0, :]
        w1 = twe_v[j, 1, :]
        for c in range(D_MODEL // LANES):
            sl = pl.ds(c * LANES, LANES)
            a_v[j, sl] = a_v[j, sl] * w0 + b_v[j, sl] * w1
        return carry

    lax.fori_loop(0, TPW, row, 0)
    pltpu.sync_copy(a_v, out_hbm.at[pl.ds(base, TPW)])


def _sc_combine(ys, slot3, twe):
    mesh = plsc.VectorSubcoreMesh(core_axis_name="c", subcore_axis_name="s")
    return pl.kernel(
        _combine_body,
        mesh=mesh,
        out_type=jax.ShapeDtypeStruct((T, D_MODEL), jnp.float32),
        scratch_types=[
            pltpu.VMEM((TOPK, TPW), jnp.int32),
            pltpu.VMEM((TPW, TOPK, LANES), jnp.float32),
            pltpu.VMEM((TPW, D_MODEL), jnp.float32),
            pltpu.VMEM((TPW, D_MODEL), jnp.float32),
            pltpu.SemaphoreType.DMA,
        ],
    )(ys, slot3, twe)


def _gmm_body(te_ref, nv_ref, xg_ref, w13_ref, s13_ref, w2_ref, s2_ref,
              ys_ref, w13d_ref, w2d_ref):
    t = pl.program_id(0)

    @pl.when(t < nv_ref[0])
    def _run():
        changed = jnp.logical_or(
            t == 0, te_ref[t] != te_ref[jnp.maximum(t - 1, 0)])

        @pl.when(changed)
        def _dequant():
            for kb in range(KB13):
                sl = pl.ds(kb * BLK, BLK)
                w13d_ref[:, sl] = (w13_ref[0, :, sl]
                                   * s13_ref[0, kb, :][:, None]).astype(jnp.bfloat16)
            for kb in range(KB2):
                sl = pl.ds(kb * BLK, BLK)
                w2d_ref[:, sl] = (w2_ref[0, :, sl]
                                  * s2_ref[0, kb, :][:, None]).astype(jnp.bfloat16)

        xt = xg_ref[...].astype(jnp.bfloat16)
        h = lax.dot_general(xt, w13d_ref[...], (((1,), (1,)), ((), ())),
                            preferred_element_type=jnp.float32)
        gate = h[:, :D_FF]
        up = h[:, D_FF:]
        act = (gate / (1.0 + jnp.exp(-gate)) * up).astype(jnp.bfloat16)
        ys_ref[...] = lax.dot_general(act, w2d_ref[...], (((1,), (1,)), ((), ())),
                                      preferred_element_type=jnp.float32)


def _tc_gmm(te, nv, xg, w13, s13e, w2, s2e):
    return pl.pallas_call(
        _gmm_body,
        grid_spec=pltpu.PrefetchScalarGridSpec(
            num_scalar_prefetch=2,
            grid=(MAX_TILES,),
            in_specs=[
                pl.BlockSpec((TILE_M, D_MODEL), lambda t, te, nv: (t, 0)),
                pl.BlockSpec((1, 2 * D_FF, D_MODEL),
                             lambda t, te, nv: (te[t], 0, 0)),
                pl.BlockSpec((1, KB13, 2 * D_FF),
                             lambda t, te, nv: (te[t], 0, 0)),
                pl.BlockSpec((1, D_MODEL, D_FF),
                             lambda t, te, nv: (te[t], 0, 0)),
                pl.BlockSpec((1, KB2, D_MODEL),
                             lambda t, te, nv: (te[t], 0, 0)),
            ],
            out_specs=pl.BlockSpec((TILE_M, D_MODEL), lambda t, te, nv: (t, 0)),
            scratch_shapes=[
                pltpu.VMEM((2 * D_FF, D_MODEL), jnp.bfloat16),
                pltpu.VMEM((D_MODEL, D_FF), jnp.bfloat16),
            ],
        ),
        out_shape=jax.ShapeDtypeStruct((MAX_ROWS, D_MODEL), jnp.float32),
    )(te, nv, xg, w13, s13e, w2, s2e)


@jax.jit
def _moe_routed(x, topk_ids, topk_weights, w13_fp8, s13e, w2_fp8, s2e):
    CH = 128
    NCH = A // CH
    e_flat = topk_ids.reshape(-1)
    oh3 = (e_flat.reshape(NCH, CH, 1)
           == jnp.arange(E, dtype=jnp.int32)).astype(jnp.float32)
    tri = jnp.tril(jnp.ones((CH, CH), jnp.float32), -1)
    local = lax.dot_general(tri, oh3, (((1,), (1,)), ((), ())),
                            preferred_element_type=jnp.float32)
    local = local.transpose(1, 0, 2)
    chunk_tot = jnp.sum(oh3, axis=1)
    chunk_off = jnp.cumsum(chunk_tot, axis=0) - chunk_tot
    rankf = local + chunk_off[:, None, :]
    rank = jnp.sum(rankf * oh3, axis=2).reshape(A).astype(jnp.int32)
    counts = jnp.sum(chunk_tot, axis=0).astype(jnp.int32)
    ntiles_e = (counts + TILE_M - 1) // TILE_M
    tile_end = jnp.cumsum(ntiles_e)
    row_off = (tile_end - ntiles_e) * TILE_M
    slot = (row_off[e_flat] + rank).reshape(T, TOPK)
    n_valid = tile_end[-1]
    tidx = jnp.arange(MAX_TILES, dtype=jnp.int32)
    te_raw = jnp.sum((tidx[:, None] >= tile_end[None, :]).astype(jnp.int32),
                     axis=1)
    last_e = te_raw[jnp.maximum(n_valid - 1, 0)]
    te = jnp.where(tidx < n_valid, te_raw, last_e).astype(jnp.int32)
    nv = n_valid.astype(jnp.int32).reshape(1)

    slot3 = slot.reshape(NW, TPW, TOPK).transpose(0, 2, 1)
    twe = jnp.broadcast_to(topk_weights[:, :, None], (T, TOPK, LANES))

    xg = _sc_dispatch(x, slot3)
    ys = _tc_gmm(te, nv, xg, w13_fp8, s13e, w2_fp8, s2e)
    return _sc_combine(ys, slot3, jnp.asarray(twe, jnp.float32))


def kernel(x, topk_ids, topk_weights, moe_n_slice, n_expert_slice, ep_shift,
           w13_fp8, w13_scale_inv, w2_fp8, w2_scale_inv):
    s13e = jnp.repeat(w13_scale_inv.transpose(0, 2, 1), BLK, axis=2)
    s2e = jnp.repeat(w2_scale_inv.transpose(0, 2, 1), BLK, axis=2)
    return _moe_routed(x, topk_ids.astype(jnp.int32),
                       topk_weights.astype(jnp.float32),
                       w13_fp8, s13e, w2_fp8, s2e)
